# Initial kernel scaffold; baseline (speedup 1.0000x reference)
import functools

import jax
import jax.numpy as jnp
from jax import lax
from jax.experimental import pallas as pl
from jax.experimental.pallas import tpu as pltpu
from jax.experimental.pallas import tpu_sc as plsc

N = 10000
E = 320000
D_IN = 128
D_HID = 16
D_OUT = 16

NC = 2
NS = 16
NW = NC * NS

B_IDX = 128
T = -(-E // (NW * B_IDX))
E_TILE = T * B_IDX
E_PAD = E_TILE * NW
N_TAB = 10240

R_TC = 1250

_MESH = plsc.VectorSubcoreMesh(core_axis_name="c", subcore_axis_name="s")


@functools.partial(
    pl.kernel,
    out_type=jax.ShapeDtypeStruct((NC, N_TAB), jnp.float32),
    mesh=_MESH,
    scratch_types=[
        pltpu.VMEM((T, B_IDX), jnp.int32),
        pltpu.VMEM((B_IDX,), jnp.float32),
        pltpu.VMEM_SHARED((N_TAB,), jnp.float32),
    ],
)
def _deg_kernel(dst_hbm, zeros_hbm, cnt_out, idx_v, ones_v, cnt_sh):
    c = lax.axis_index("c")
    s = lax.axis_index("s")

    @pl.when(s == 0)
    def _():
        pltpu.sync_copy(zeros_hbm, cnt_sh)

    pltpu.sync_copy(dst_hbm.at[c, s], idx_v)
    for k in range(B_IDX // 16):
        ones_v[pl.ds(k * 16, 16)] = jnp.full((16,), 1.0, jnp.float32)
    plsc.subcore_barrier()

    def chunk(j, carry):
        pltpu.sync_copy(ones_v, cnt_sh.at[idx_v.at[j]], add=True)
        return carry

    lax.fori_loop(0, T, chunk, 0)
    plsc.subcore_barrier()

    @pl.when(s == 0)
    def _():
        pltpu.sync_copy(cnt_sh, cnt_out.at[c])


@functools.partial(
    pl.kernel,
    out_type=jax.ShapeDtypeStruct((NC, N_TAB, D_HID), jnp.float32),
    mesh=_MESH,
    scratch_types=[
        pltpu.VMEM((T, B_IDX), jnp.int32),
        pltpu.VMEM((T, B_IDX), jnp.int32),
        pltpu.VMEM((2, B_IDX, D_HID), jnp.float32),
        pltpu.VMEM_SHARED((N_TAB, D_HID), jnp.float32),
        pltpu.SemaphoreType.DMA,
        pltpu.SemaphoreType.DMA,
    ],
)
def _msg_kernel(src_hbm, dst_hbm, z_hbm, zeros_hbm, acc_out,
                sidx_v, didx_v, rows_v, acc_sh, sem0, sem1):
    c = lax.axis_index("c")
    s = lax.axis_index("s")

    @pl.when(s == 0)
    def _():
        pltpu.sync_copy(zeros_hbm, acc_sh)

    pltpu.sync_copy(src_hbm.at[c, s], sidx_v)
    pltpu.sync_copy(dst_hbm.at[c, s], didx_v)
    plsc.subcore_barrier()

    sems = (sem0, sem1)
    pltpu.async_copy(z_hbm.at[sidx_v.at[0]], rows_v.at[0], sems[0])

    def chunk(j, carry):
        for b in range(2):
            @pl.when(lax.rem(j, 2) == b)
            def _():
                pltpu.make_async_copy(
                    z_hbm.at[sidx_v.at[j]], rows_v.at[b], sems[b]).wait()

                @pl.when(j + 1 < T)
                def _():
                    pltpu.async_copy(
                        z_hbm.at[sidx_v.at[j + 1]], rows_v.at[1 - b],
                        sems[1 - b])

                pltpu.sync_copy(rows_v.at[b], acc_sh.at[didx_v.at[j]],
                                add=True)
        return carry

    lax.fori_loop(0, T, chunk, 0)
    plsc.subcore_barrier()

    @pl.when(s == 0)
    def _():
        pltpu.sync_copy(acc_sh, acc_out.at[c])


def _mm_body(x_ref, w1_ref, c0_ref, c1_ref, z_ref, dinv_ref):
    deg = 1.0 + c0_ref[...] + c1_ref[...]
    dinv = lax.rsqrt(deg)
    xw = jnp.dot(x_ref[...], w1_ref[...], preferred_element_type=jnp.float32)
    z_ref[...] = xw * dinv
    dinv_ref[...] = dinv


def _mm_call(x, w1, c0, c1):
    grid = (N // R_TC,)
    return pl.pallas_call(
        _mm_body,
        grid=grid,
        in_specs=[
            pl.BlockSpec((R_TC, D_IN), lambda i: (i, 0)),
            pl.BlockSpec((D_IN, D_HID), lambda i: (0, 0)),
            pl.BlockSpec((R_TC, 1), lambda i: (i, 0)),
            pl.BlockSpec((R_TC, 1), lambda i: (i, 0)),
        ],
        out_specs=[
            pl.BlockSpec((R_TC, D_HID), lambda i: (i, 0)),
            pl.BlockSpec((R_TC, 1), lambda i: (i, 0)),
        ],
        out_shape=[
            jax.ShapeDtypeStruct((N, D_HID), jnp.float32),
            jax.ShapeDtypeStruct((N, 1), jnp.float32),
        ],
    )(x, w1, c0, c1)


def _ep_body(a0_ref, a1_ref, z_ref, dinv_ref, b1_ref, w2_ref, b2_ref,
             out_ref):
    t = a0_ref[...] + a1_ref[...] + z_ref[...]
    h = dinv_ref[...] * t + b1_ref[...]
    h = jnp.maximum(h, 0.0)
    y = jnp.dot(h, w2_ref[...], preferred_element_type=jnp.float32)
    y = y + b2_ref[...]
    m = jnp.max(y, axis=-1, keepdims=True)
    e = jnp.exp(y - m)
    lse = jnp.log(jnp.sum(e, axis=-1, keepdims=True)) + m
    out_ref[...] = y - lse


def _ep_call(a0, a1, z, dinv, b1, w2, b2):
    grid = (N // R_TC,)
    rspec = pl.BlockSpec((R_TC, D_HID), lambda i: (i, 0))
    return pl.pallas_call(
        _ep_body,
        grid=grid,
        in_specs=[
            rspec, rspec, rspec,
            pl.BlockSpec((R_TC, 1), lambda i: (i, 0)),
            pl.BlockSpec((1, D_HID), lambda i: (0, 0)),
            pl.BlockSpec((D_HID, D_OUT), lambda i: (0, 0)),
            pl.BlockSpec((1, D_OUT), lambda i: (0, 0)),
        ],
        out_specs=pl.BlockSpec((R_TC, D_OUT), lambda i: (i, 0)),
        out_shape=jax.ShapeDtypeStruct((N, D_OUT), jnp.float32),
    )(a0, a1, z, dinv, b1, w2, b2)


def kernel(x, edge_index, W1, b1, W2, b2):
    src = edge_index[0]
    dst = edge_index[1]
    pad = E_PAD - E
    src_p = jnp.concatenate([src, jnp.zeros((pad,), jnp.int32)])
    dst_p = jnp.concatenate([dst, jnp.full((pad,), N, jnp.int32)])
    src_r = src_p.reshape(NC, NS, T, B_IDX)
    dst_r = dst_p.reshape(NC, NS, T, B_IDX)

    zeros_n = jnp.zeros((N_TAB,), jnp.float32)
    zeros_nh = jnp.zeros((N_TAB, D_HID), jnp.float32)

    cnt = _deg_kernel(dst_r, zeros_n)
    c0 = cnt[0, :N, None]
    c1 = cnt[1, :N, None]
    z, dinv = _mm_call(x, W1, c0, c1)

    acc = _msg_kernel(src_r, dst_r, z, zeros_nh)
    a0 = acc[0, :N, :]
    a1 = acc[1, :N, :]
    return _ep_call(a0, a1, z, dinv, b1.reshape(1, D_HID), W2,
                    b2.reshape(1, D_OUT))


# speedup vs baseline: 42.2994x; 42.2994x over previous
import functools

import jax
import jax.numpy as jnp
from jax import lax
from jax.experimental import pallas as pl
from jax.experimental.pallas import tpu as pltpu
from jax.experimental.pallas import tpu_sc as plsc

N = 10000
E = 320000
D_IN = 128
D_HID = 16
D_OUT = 16

NC = 2
NS = 16
NW = NC * NS

B_IDX = 128
T = -(-E // (NW * B_IDX))
E_TILE = T * B_IDX
E_PAD = E_TILE * NW
N_TAB = 10240

R_TC = 1000

def _deg_body(dst_hbm, zeros_hbm, cnt_out, idx_v, ones_v, cnt_sh):
    c = lax.axis_index("c")
    s = lax.axis_index("s")

    @pl.when(s == 0)
    def _():
        pltpu.sync_copy(zeros_hbm, cnt_sh)

    pltpu.sync_copy(dst_hbm.at[c, s], idx_v)
    for k in range(B_IDX // 16):
        ones_v[pl.ds(k * 16, 16)] = jnp.full((16,), 1.0, jnp.float32)
    plsc.subcore_barrier()

    def chunk(j, carry):
        pltpu.sync_copy(ones_v, cnt_sh.at[idx_v.at[j]], add=True)
        return carry

    lax.fori_loop(0, T, chunk, 0)
    plsc.subcore_barrier()

    @pl.when(s == 0)
    def _():
        pltpu.sync_copy(cnt_sh, cnt_out.at[c])


def _msg_body(src_hbm, dst_hbm, z_hbm, zeros_hbm, acc_out,
              sidx_v, didx_v, rows_v, acc_sh, sem0, sem1):
    c = lax.axis_index("c")
    s = lax.axis_index("s")

    @pl.when(s == 0)
    def _():
        pltpu.sync_copy(zeros_hbm, acc_sh)

    pltpu.sync_copy(src_hbm.at[c, s], sidx_v)
    pltpu.sync_copy(dst_hbm.at[c, s], didx_v)
    plsc.subcore_barrier()

    sems = (sem0, sem1)
    pltpu.async_copy(z_hbm.at[sidx_v.at[0]], rows_v.at[0], sems[0])

    def chunk(j, carry):
        for b in range(2):
            @pl.when(lax.rem(j, 2) == b)
            def _():
                pltpu.make_async_copy(
                    z_hbm.at[sidx_v.at[j]], rows_v.at[b], sems[b]).wait()

                @pl.when(j + 1 < T)
                def _():
                    pltpu.async_copy(
                        z_hbm.at[sidx_v.at[j + 1]], rows_v.at[1 - b],
                        sems[1 - b])

                pltpu.sync_copy(rows_v.at[b], acc_sh.at[didx_v.at[j]],
                                add=True)
        return carry

    lax.fori_loop(0, T, chunk, 0)
    plsc.subcore_barrier()

    @pl.when(s == 0)
    def _():
        pltpu.sync_copy(acc_sh, acc_out.at[c])


@functools.lru_cache(maxsize=None)
def _sc_kernels():
    mesh = plsc.VectorSubcoreMesh(core_axis_name="c", subcore_axis_name="s",
                                  num_cores=NC, num_subcores=NS)
    deg = pl.kernel(
        _deg_body,
        out_type=jax.ShapeDtypeStruct((NC, N_TAB), jnp.float32),
        mesh=mesh,
        scratch_types=[
            pltpu.VMEM((T, B_IDX), jnp.int32),
            pltpu.VMEM((B_IDX,), jnp.float32),
            pltpu.VMEM_SHARED((N_TAB,), jnp.float32),
        ],
    )
    msg = pl.kernel(
        _msg_body,
        out_type=jax.ShapeDtypeStruct((NC, N_TAB, D_HID), jnp.float32),
        mesh=mesh,
        compiler_params=pltpu.CompilerParams(use_tc_tiling_on_sc=False),
        scratch_types=[
            pltpu.VMEM((T, B_IDX), jnp.int32),
            pltpu.VMEM((T, B_IDX), jnp.int32),
            pltpu.VMEM((2, B_IDX, D_HID), jnp.float32),
            pltpu.VMEM_SHARED((N_TAB, D_HID), jnp.float32),
            pltpu.SemaphoreType.DMA,
            pltpu.SemaphoreType.DMA,
        ],
    )
    return deg, msg


def _mm_body(x_ref, w1_ref, c0_ref, c1_ref, z_ref, dinv_ref):
    deg = 1.0 + c0_ref[...] + c1_ref[...]
    dinv = lax.rsqrt(deg)
    xw = jnp.dot(x_ref[...], w1_ref[...], preferred_element_type=jnp.float32)
    z_ref[...] = xw * dinv
    dinv_ref[...] = dinv


def _mm_call(x, w1, c0, c1):
    grid = (N // R_TC,)
    return pl.pallas_call(
        _mm_body,
        grid=grid,
        in_specs=[
            pl.BlockSpec((R_TC, D_IN), lambda i: (i, 0)),
            pl.BlockSpec((D_IN, D_HID), lambda i: (0, 0)),
            pl.BlockSpec((R_TC, 1), lambda i: (i, 0)),
            pl.BlockSpec((R_TC, 1), lambda i: (i, 0)),
        ],
        out_specs=[
            pl.BlockSpec((R_TC, D_HID), lambda i: (i, 0)),
            pl.BlockSpec((R_TC, 1), lambda i: (i, 0)),
        ],
        out_shape=[
            jax.ShapeDtypeStruct((N, D_HID), jnp.float32),
            jax.ShapeDtypeStruct((N, 1), jnp.float32),
        ],
    )(x, w1, c0, c1)


def _ep_body(a0_ref, a1_ref, z_ref, dinv_ref, b1_ref, w2_ref, b2_ref,
             out_ref):
    t = a0_ref[...] + a1_ref[...] + z_ref[...]
    h = dinv_ref[...] * t + b1_ref[...]
    h = jnp.maximum(h, 0.0)
    y = jnp.dot(h, w2_ref[...], preferred_element_type=jnp.float32)
    y = y + b2_ref[...]
    m = jnp.max(y, axis=-1, keepdims=True)
    e = jnp.exp(y - m)
    lse = jnp.log(jnp.sum(e, axis=-1, keepdims=True)) + m
    out_ref[...] = y - lse


def _ep_call(a0, a1, z, dinv, b1, w2, b2):
    grid = (N // R_TC,)
    rspec = pl.BlockSpec((R_TC, D_HID), lambda i: (i, 0))
    return pl.pallas_call(
        _ep_body,
        grid=grid,
        in_specs=[
            rspec, rspec, rspec,
            pl.BlockSpec((R_TC, 1), lambda i: (i, 0)),
            pl.BlockSpec((1, D_HID), lambda i: (0, 0)),
            pl.BlockSpec((D_HID, D_OUT), lambda i: (0, 0)),
            pl.BlockSpec((1, D_OUT), lambda i: (0, 0)),
        ],
        out_specs=pl.BlockSpec((R_TC, D_OUT), lambda i: (i, 0)),
        out_shape=jax.ShapeDtypeStruct((N, D_OUT), jnp.float32),
    )(a0, a1, z, dinv, b1, w2, b2)


def kernel(x, edge_index, W1, b1, W2, b2):
    src = edge_index[0]
    dst = edge_index[1]
    pad = E_PAD - E
    src_p = jnp.concatenate([src, jnp.zeros((pad,), jnp.int32)])
    dst_p = jnp.concatenate([dst, jnp.full((pad,), N, jnp.int32)])
    src_r = src_p.reshape(NC, NS, T, B_IDX)
    dst_r = dst_p.reshape(NC, NS, T, B_IDX)

    zeros_n = jnp.zeros((N_TAB,), jnp.float32)
    zeros_nh = jnp.zeros((N_TAB, D_HID), jnp.float32)

    deg_kernel, msg_kernel = _sc_kernels()
    cnt = deg_kernel(dst_r, zeros_n)
    c0 = cnt[0, :N, None]
    c1 = cnt[1, :N, None]
    z, dinv = _mm_call(x, W1, c0, c1)

    acc = msg_kernel(src_r, dst_r, z, zeros_nh)
    a0 = acc[0, :N, :]
    a1 = acc[1, :N, :]
    return _ep_call(a0, a1, z, dinv, b1.reshape(1, D_HID), W2,
                    b2.reshape(1, D_OUT))


# speedup vs baseline: 44.9872x; 1.0635x over previous
import functools

import jax
import jax.numpy as jnp
from jax import lax
from jax.experimental import pallas as pl
from jax.experimental.pallas import tpu as pltpu
from jax.experimental.pallas import tpu_sc as plsc

N = 10000
E = 320000
D_IN = 128
D_HID = 16
D_OUT = 16

NC = 2
NS = 16
NW = NC * NS

B_IDX = 128
G_CHUNK = 1024
T_G = 10
E_TILE = T_G * G_CHUNK
SUBT = E_TILE // B_IDX
SUB = G_CHUNK // B_IDX
T = SUBT
E_PAD = E_TILE * NW
N_TAB = 10240

R_TC = 1000

def _deg_body(dst_hbm, zeros_hbm, cnt_out, idx_v, ones_v, cnt_sh, sem):
    c = lax.axis_index("c")
    s = lax.axis_index("s")

    @pl.when(s == 0)
    def _():
        pltpu.sync_copy(zeros_hbm, cnt_sh)

    pltpu.sync_copy(dst_hbm.at[c, s], idx_v)
    for k in range(B_IDX // 16):
        ones_v[pl.ds(k * 16, 16)] = jnp.full((16,), 1.0, jnp.float32)
    plsc.subcore_barrier()

    def chunk(j, carry):
        pltpu.async_copy(ones_v, cnt_sh.at[idx_v.at[j]], sem, add=True)
        return carry

    lax.fori_loop(0, T, chunk, 0)

    def drain(j, carry):
        pltpu.make_async_copy(ones_v, cnt_sh.at[idx_v.at[j]], sem).wait()
        return carry

    lax.fori_loop(0, T, drain, 0)
    plsc.subcore_barrier()

    @pl.when(s == 0)
    def _():
        pltpu.sync_copy(cnt_sh, cnt_out.at[c])


def _msg_body(src_hbm, dst_hbm, z_hbm, zeros_hbm, acc_out,
              sidx_v, didx_v, rows_v, acc_sh, gsem0, gsem1, ssem0, ssem1):
    c = lax.axis_index("c")
    s = lax.axis_index("s")

    @pl.when(s == 0)
    def _():
        pltpu.sync_copy(zeros_hbm, acc_sh)

    pltpu.sync_copy(src_hbm.at[c, s], sidx_v)
    pltpu.sync_copy(dst_hbm.at[c, s], didx_v)
    plsc.subcore_barrier()

    gsems = (gsem0, gsem1)
    ssems = (ssem0, ssem1)
    pltpu.async_copy(z_hbm.at[sidx_v.at[0]], rows_v.at[0], gsems[0])

    def chunk(g, carry):
        for b in range(2):
            @pl.when(lax.rem(g, 2) == b)
            def _():
                pltpu.make_async_copy(
                    z_hbm.at[sidx_v.at[g]], rows_v.at[b], gsems[b]).wait()

                @pl.when(g >= 1)
                def _():
                    pltpu.make_async_copy(
                        z_hbm.at[pl.ds(0, G_CHUNK)], rows_v.at[1 - b],
                        ssems[1 - b]).wait()

                @pl.when(g + 1 < T_G)
                def _():
                    pltpu.async_copy(
                        z_hbm.at[sidx_v.at[g + 1]], rows_v.at[1 - b],
                        gsems[1 - b])

                for k in range(SUB):
                    pltpu.async_copy(
                        rows_v.at[b].at[pl.ds(k * B_IDX, B_IDX)],
                        acc_sh.at[didx_v.at[g * SUB + k]], ssems[b],
                        add=True)
        return carry

    lax.fori_loop(0, T_G, chunk, 0)
    last = (T_G - 1) % 2
    pltpu.make_async_copy(z_hbm.at[pl.ds(0, G_CHUNK)], rows_v.at[last],
                          ssems[last]).wait()
    plsc.subcore_barrier()

    @pl.when(s == 0)
    def _():
        pltpu.sync_copy(acc_sh, acc_out.at[c])


@functools.lru_cache(maxsize=None)
def _sc_kernels():
    mesh = plsc.VectorSubcoreMesh(core_axis_name="c", subcore_axis_name="s",
                                  num_cores=NC, num_subcores=NS)
    deg = pl.kernel(
        _deg_body,
        out_type=jax.ShapeDtypeStruct((NC, N_TAB), jnp.float32),
        mesh=mesh,
        scratch_types=[
            pltpu.VMEM((T, B_IDX), jnp.int32),
            pltpu.VMEM((B_IDX,), jnp.float32),
            pltpu.VMEM_SHARED((N_TAB,), jnp.float32),
            pltpu.SemaphoreType.DMA,
        ],
    )
    msg = pl.kernel(
        _msg_body,
        out_type=jax.ShapeDtypeStruct((NC, N_TAB, D_HID), jnp.float32),
        mesh=mesh,
        compiler_params=pltpu.CompilerParams(use_tc_tiling_on_sc=False),
        scratch_types=[
            pltpu.VMEM((T_G, G_CHUNK), jnp.int32),
            pltpu.VMEM((SUBT, B_IDX), jnp.int32),
            pltpu.VMEM((2, G_CHUNK, D_HID), jnp.float32),
            pltpu.VMEM_SHARED((N_TAB, D_HID), jnp.float32),
            pltpu.SemaphoreType.DMA,
            pltpu.SemaphoreType.DMA,
            pltpu.SemaphoreType.DMA,
            pltpu.SemaphoreType.DMA,
        ],
    )
    return deg, msg


def _mm_body(x_ref, w1_ref, c0_ref, c1_ref, z_ref, dinv_ref):
    deg = 1.0 + c0_ref[...] + c1_ref[...]
    dinv = lax.rsqrt(deg)
    xw = jnp.dot(x_ref[...], w1_ref[...], preferred_element_type=jnp.float32)
    z_ref[...] = xw * dinv
    dinv_ref[...] = dinv


def _mm_call(x, w1, c0, c1):
    grid = (N // R_TC,)
    return pl.pallas_call(
        _mm_body,
        grid=grid,
        in_specs=[
            pl.BlockSpec((R_TC, D_IN), lambda i: (i, 0)),
            pl.BlockSpec((D_IN, D_HID), lambda i: (0, 0)),
            pl.BlockSpec((R_TC, 1), lambda i: (i, 0)),
            pl.BlockSpec((R_TC, 1), lambda i: (i, 0)),
        ],
        out_specs=[
            pl.BlockSpec((R_TC, D_HID), lambda i: (i, 0)),
            pl.BlockSpec((R_TC, 1), lambda i: (i, 0)),
        ],
        out_shape=[
            jax.ShapeDtypeStruct((N, D_HID), jnp.float32),
            jax.ShapeDtypeStruct((N, 1), jnp.float32),
        ],
    )(x, w1, c0, c1)


def _ep_body(a0_ref, a1_ref, z_ref, dinv_ref, b1_ref, w2_ref, b2_ref,
             out_ref):
    t = a0_ref[...] + a1_ref[...] + z_ref[...]
    h = dinv_ref[...] * t + b1_ref[...]
    h = jnp.maximum(h, 0.0)
    y = jnp.dot(h, w2_ref[...], preferred_element_type=jnp.float32)
    y = y + b2_ref[...]
    m = jnp.max(y, axis=-1, keepdims=True)
    e = jnp.exp(y - m)
    lse = jnp.log(jnp.sum(e, axis=-1, keepdims=True)) + m
    out_ref[...] = y - lse


def _ep_call(a0, a1, z, dinv, b1, w2, b2):
    grid = (N // R_TC,)
    rspec = pl.BlockSpec((R_TC, D_HID), lambda i: (i, 0))
    return pl.pallas_call(
        _ep_body,
        grid=grid,
        in_specs=[
            rspec, rspec, rspec,
            pl.BlockSpec((R_TC, 1), lambda i: (i, 0)),
            pl.BlockSpec((1, D_HID), lambda i: (0, 0)),
            pl.BlockSpec((D_HID, D_OUT), lambda i: (0, 0)),
            pl.BlockSpec((1, D_OUT), lambda i: (0, 0)),
        ],
        out_specs=pl.BlockSpec((R_TC, D_OUT), lambda i: (i, 0)),
        out_shape=jax.ShapeDtypeStruct((N, D_OUT), jnp.float32),
    )(a0, a1, z, dinv, b1, w2, b2)


def kernel(x, edge_index, W1, b1, W2, b2):
    src = edge_index[0]
    dst = edge_index[1]
    pad = E_PAD - E
    src_p = jnp.concatenate([src, jnp.zeros((pad,), jnp.int32)])
    dst_p = jnp.concatenate([dst, jnp.full((pad,), N, jnp.int32)])
    src_r = src_p.reshape(NC, NS, T_G, G_CHUNK)
    dst_r = dst_p.reshape(NC, NS, SUBT, B_IDX)

    zeros_n = jnp.zeros((N_TAB,), jnp.float32)
    zeros_nh = jnp.zeros((N_TAB, D_HID), jnp.float32)

    deg_kernel, msg_kernel = _sc_kernels()
    cnt = deg_kernel(dst_r, zeros_n)
    c0 = cnt[0, :N, None]
    c1 = cnt[1, :N, None]
    z, dinv = _mm_call(x, W1, c0, c1)

    acc = msg_kernel(src_r, dst_r, z, zeros_nh)
    a0 = acc[0, :N, :]
    a1 = acc[1, :N, :]
    return _ep_call(a0, a1, z, dinv, b1.reshape(1, D_HID), W2,
                    b2.reshape(1, D_OUT))


# speedup vs baseline: 47.6250x; 1.0586x over previous
import functools

import jax
import jax.numpy as jnp
from jax import lax
from jax.experimental import pallas as pl
from jax.experimental.pallas import tpu as pltpu
from jax.experimental.pallas import tpu_sc as plsc

N = 10000
E = 320000
D_IN = 128
D_HID = 16
D_OUT = 16

NC = 2
NS = 16
NW = NC * NS

B_IDX = 128
G_CHUNK = 1024
T_G = 10
E_TILE = T_G * G_CHUNK
SUBT = E_TILE // B_IDX
SUB = G_CHUNK // B_IDX
T = SUBT
E_PAD = E_TILE * NW
N_TAB = 10240

R_TC = 1000

def _deg_body(dst_hbm, zeros_hbm, cnt_out, idx_v, ones_v, cnt_sh, sem):
    c = lax.axis_index("c")
    s = lax.axis_index("s")

    @pl.when(s == 0)
    def _():
        pltpu.sync_copy(zeros_hbm, cnt_sh)

    pltpu.sync_copy(dst_hbm.at[c, s], idx_v)
    for k in range(B_IDX // 16):
        ones_v[pl.ds(k * 16, 16)] = jnp.full((16,), 1.0, jnp.float32)
    plsc.subcore_barrier()

    def chunk(j, carry):
        pltpu.async_copy(ones_v, cnt_sh.at[idx_v.at[j]], sem, add=True)
        return carry

    lax.fori_loop(0, T, chunk, 0)

    def drain(j, carry):
        pltpu.make_async_copy(ones_v, cnt_sh.at[idx_v.at[j]], sem).wait()
        return carry

    lax.fori_loop(0, T, drain, 0)
    plsc.subcore_barrier()

    @pl.when(s == 0)
    def _():
        pltpu.sync_copy(cnt_sh, cnt_out.at[c])


def _msg_body(src_hbm, dst_hbm, z_hbm, zeros_hbm, acc_out,
              sidx_v, didx_v, rows_v, acc_sh, gsem0, gsem1, ssem0, ssem1):
    c = lax.axis_index("c")
    s = lax.axis_index("s")

    @pl.when(s == 0)
    def _():
        pltpu.sync_copy(zeros_hbm, acc_sh)

    pltpu.sync_copy(src_hbm.at[c, s], sidx_v)
    pltpu.sync_copy(dst_hbm.at[c, s], didx_v)
    plsc.subcore_barrier()

    gsems = (gsem0, gsem1)
    ssems = (ssem0, ssem1)
    pltpu.async_copy(z_hbm.at[sidx_v.at[0]], rows_v.at[0], gsems[0])

    def chunk(g, carry):
        for b in range(2):
            @pl.when(lax.rem(g, 2) == b)
            def _():
                pltpu.make_async_copy(
                    z_hbm.at[sidx_v.at[g]], rows_v.at[b], gsems[b]).wait()

                @pl.when(g >= 1)
                def _():
                    pltpu.make_async_copy(
                        z_hbm.at[pl.ds(0, G_CHUNK)], rows_v.at[1 - b],
                        ssems[1 - b]).wait()

                @pl.when(g + 1 < T_G)
                def _():
                    pltpu.async_copy(
                        z_hbm.at[sidx_v.at[g + 1]], rows_v.at[1 - b],
                        gsems[1 - b])

                for k in range(SUB):
                    pltpu.async_copy(
                        rows_v.at[b].at[pl.ds(k * B_IDX, B_IDX)],
                        acc_sh.at[didx_v.at[g * SUB + k]], ssems[b],
                        add=True)
        return carry

    lax.fori_loop(0, T_G, chunk, 0)
    last = (T_G - 1) % 2
    pltpu.make_async_copy(z_hbm.at[pl.ds(0, G_CHUNK)], rows_v.at[last],
                          ssems[last]).wait()
    plsc.subcore_barrier()

    @pl.when(s == 0)
    def _():
        pltpu.sync_copy(acc_sh, acc_out.at[c])


@functools.lru_cache(maxsize=None)
def _sc_kernels():
    mesh = plsc.VectorSubcoreMesh(core_axis_name="c", subcore_axis_name="s",
                                  num_cores=NC, num_subcores=NS)
    deg = pl.kernel(
        _deg_body,
        out_type=jax.ShapeDtypeStruct((NC, N_TAB), jnp.float32),
        mesh=mesh,
        scratch_types=[
            pltpu.VMEM((T, B_IDX), jnp.int32),
            pltpu.VMEM((B_IDX,), jnp.float32),
            pltpu.VMEM_SHARED((N_TAB,), jnp.float32),
            pltpu.SemaphoreType.DMA,
        ],
    )
    msg = pl.kernel(
        _msg_body,
        out_type=jax.ShapeDtypeStruct((NC, N_TAB, D_HID), jnp.float32),
        mesh=mesh,
        compiler_params=pltpu.CompilerParams(use_tc_tiling_on_sc=False),
        scratch_types=[
            pltpu.VMEM((T_G, G_CHUNK), jnp.int32),
            pltpu.VMEM((SUBT, B_IDX), jnp.int32),
            pltpu.VMEM((2, G_CHUNK, D_HID), jnp.float32),
            pltpu.VMEM_SHARED((N_TAB, D_HID), jnp.float32),
            pltpu.SemaphoreType.DMA,
            pltpu.SemaphoreType.DMA,
            pltpu.SemaphoreType.DMA,
            pltpu.SemaphoreType.DMA,
        ],
    )
    return deg, msg


def _mm_body(x_ref, w1_ref, c0_ref, c1_ref, z_ref, dinv_ref):
    deg = 1.0 + c0_ref[...] + c1_ref[...]
    dinv = lax.rsqrt(deg)
    xw = jnp.dot(x_ref[...], w1_ref[...], preferred_element_type=jnp.float32)
    z_ref[...] = xw * dinv
    dinv_ref[...] = dinv


def _mm_call(x, w1, c0, c1):
    grid = (N // R_TC,)
    return pl.pallas_call(
        _mm_body,
        grid=grid,
        in_specs=[
            pl.BlockSpec((R_TC, D_IN), lambda i: (i, 0)),
            pl.BlockSpec((D_IN, D_HID), lambda i: (0, 0)),
            pl.BlockSpec((R_TC, 1), lambda i: (i, 0)),
            pl.BlockSpec((R_TC, 1), lambda i: (i, 0)),
        ],
        out_specs=[
            pl.BlockSpec((R_TC, D_HID), lambda i: (i, 0)),
            pl.BlockSpec((R_TC, 1), lambda i: (i, 0)),
        ],
        out_shape=[
            jax.ShapeDtypeStruct((N, D_HID), jnp.float32),
            jax.ShapeDtypeStruct((N, 1), jnp.float32),
        ],
    )(x, w1, c0, c1)


def _ep_body(a0_ref, a1_ref, z_ref, dinv_ref, b1_ref, w2_ref, b2_ref,
             out_ref):
    t = a0_ref[...] + a1_ref[...] + z_ref[...]
    h = dinv_ref[...] * t + b1_ref[...]
    h = jnp.maximum(h, 0.0)
    y = jnp.dot(h, w2_ref[...], preferred_element_type=jnp.float32)
    y = y + b2_ref[...]
    m = jnp.max(y, axis=-1, keepdims=True)
    e = jnp.exp(y - m)
    lse = jnp.log(jnp.sum(e, axis=-1, keepdims=True)) + m
    out_ref[...] = y - lse


def _ep_call(a0, a1, z, dinv, b1, w2, b2):
    grid = (N // R_TC,)
    rspec = pl.BlockSpec((R_TC, D_HID), lambda i: (i, 0))
    return pl.pallas_call(
        _ep_body,
        grid=grid,
        in_specs=[
            rspec, rspec, rspec,
            pl.BlockSpec((R_TC, 1), lambda i: (i, 0)),
            pl.BlockSpec((1, D_HID), lambda i: (0, 0)),
            pl.BlockSpec((D_HID, D_OUT), lambda i: (0, 0)),
            pl.BlockSpec((1, D_OUT), lambda i: (0, 0)),
        ],
        out_specs=pl.BlockSpec((R_TC, D_OUT), lambda i: (i, 0)),
        out_shape=jax.ShapeDtypeStruct((N, D_OUT), jnp.float32),
    )(a0, a1, z, dinv, b1, w2, b2)


def kernel(x, edge_index, W1, b1, W2, b2):
    src = edge_index[0]
    dst = edge_index[1]
    pad_t = E_TILE - E // NW
    pad_src = jnp.zeros((NW, pad_t), jnp.int32)
    pad_dst = jnp.broadcast_to(
        N + jnp.arange(pad_t, dtype=jnp.int32) % (N_TAB - N), (NW, pad_t))
    src_p = jnp.concatenate([src.reshape(NW, E // NW), pad_src], axis=1)
    dst_p = jnp.concatenate([dst.reshape(NW, E // NW), pad_dst], axis=1)
    src_r = src_p.reshape(NC, NS, T_G, G_CHUNK)
    dst_r = dst_p.reshape(NC, NS, SUBT, B_IDX)

    zeros_n = jnp.zeros((N_TAB,), jnp.float32)
    zeros_nh = jnp.zeros((N_TAB, D_HID), jnp.float32)

    deg_kernel, msg_kernel = _sc_kernels()
    cnt = deg_kernel(dst_r, zeros_n)
    c0 = cnt[0, :N, None]
    c1 = cnt[1, :N, None]
    z, dinv = _mm_call(x, W1, c0, c1)

    acc = msg_kernel(src_r, dst_r, z, zeros_nh)
    a0 = acc[0, :N, :]
    a1 = acc[1, :N, :]
    return _ep_call(a0, a1, z, dinv, b1.reshape(1, D_HID), W2,
                    b2.reshape(1, D_OUT))


# speedup vs baseline: 47.6641x; 1.0008x over previous
import functools

import jax
import jax.numpy as jnp
from jax import lax
from jax.experimental import pallas as pl
from jax.experimental.pallas import tpu as pltpu
from jax.experimental.pallas import tpu_sc as plsc

N = 10000
E = 320000
D_IN = 128
D_HID = 16
D_OUT = 16

NC = 2
NS = 16
NW = NC * NS

B_IDX = 128
G_CHUNK = 1024
T_G = 10
E_TILE = T_G * G_CHUNK
SUBT = E_TILE // B_IDX
SUB = G_CHUNK // B_IDX
T = SUBT
E_PAD = E_TILE * NW
N_TAB = 10240

R_TC = 1000

def _deg_body(dst_hbm, zeros_hbm, cnt_out, idx_v, ones_v, cnt_sh, sem):
    c = lax.axis_index("c")
    s = lax.axis_index("s")

    @pl.when(s == 0)
    def _():
        pltpu.sync_copy(zeros_hbm, cnt_sh)

    pltpu.sync_copy(dst_hbm.at[c, s], idx_v)
    for k in range(B_IDX // 16):
        ones_v[pl.ds(k * 16, 16)] = jnp.full((16,), 1.0, jnp.float32)
    plsc.subcore_barrier()

    def chunk(j, carry):
        pltpu.async_copy(ones_v, cnt_sh.at[idx_v.at[j]], sem, add=True)
        return carry

    lax.fori_loop(0, T, chunk, 0)

    def drain(j, carry):
        pltpu.make_async_copy(ones_v, cnt_sh.at[idx_v.at[j]], sem).wait()
        return carry

    lax.fori_loop(0, T, drain, 0)
    plsc.subcore_barrier()

    @pl.when(s == 0)
    def _():
        pltpu.sync_copy(cnt_sh, cnt_out.at[c])


def _msg_body(src_hbm, dst_hbm, z_hbm, zeros_hbm, acc_out,
              sidx_v, didx_v, rows_v, acc_sh, gsem0, gsem1, ssem0, ssem1):
    c = lax.axis_index("c")
    s = lax.axis_index("s")

    @pl.when(s == 0)
    def _():
        pltpu.sync_copy(zeros_hbm, acc_sh)

    pltpu.sync_copy(src_hbm.at[c, s], sidx_v)
    pltpu.sync_copy(dst_hbm.at[c, s], didx_v)
    plsc.subcore_barrier()

    gsems = (gsem0, gsem1)
    ssems = (ssem0, ssem1)
    pltpu.async_copy(z_hbm.at[sidx_v.at[0]], rows_v.at[0], gsems[0])

    def chunk(g, carry):
        for b in range(2):
            @pl.when(lax.rem(g, 2) == b)
            def _():
                pltpu.make_async_copy(
                    z_hbm.at[sidx_v.at[g]], rows_v.at[b], gsems[b]).wait()

                @pl.when(g >= 1)
                def _():
                    pltpu.make_async_copy(
                        z_hbm.at[pl.ds(0, G_CHUNK)], rows_v.at[1 - b],
                        ssems[1 - b]).wait()

                @pl.when(g + 1 < T_G)
                def _():
                    pltpu.async_copy(
                        z_hbm.at[sidx_v.at[g + 1]], rows_v.at[1 - b],
                        gsems[1 - b])

                for k in range(SUB):
                    pltpu.async_copy(
                        rows_v.at[b].at[pl.ds(k * B_IDX, B_IDX)],
                        acc_sh.at[didx_v.at[g * SUB + k]], ssems[b],
                        add=True)
        return carry

    lax.fori_loop(0, T_G, chunk, 0)
    last = (T_G - 1) % 2
    pltpu.make_async_copy(z_hbm.at[pl.ds(0, G_CHUNK)], rows_v.at[last],
                          ssems[last]).wait()
    plsc.subcore_barrier()

    @pl.when(s == 0)
    def _():
        pltpu.sync_copy(acc_sh, acc_out.at[c])


@functools.lru_cache(maxsize=None)
def _sc_kernels():
    mesh = plsc.VectorSubcoreMesh(core_axis_name="c", subcore_axis_name="s",
                                  num_cores=NC, num_subcores=NS)
    deg = pl.kernel(
        _deg_body,
        out_type=jax.ShapeDtypeStruct((NC, N_TAB), jnp.float32),
        mesh=mesh,
        scratch_types=[
            pltpu.VMEM((T, B_IDX), jnp.int32),
            pltpu.VMEM((B_IDX,), jnp.float32),
            pltpu.VMEM_SHARED((N_TAB,), jnp.float32),
            pltpu.SemaphoreType.DMA,
        ],
    )
    msg = pl.kernel(
        _msg_body,
        out_type=jax.ShapeDtypeStruct((NC, N_TAB, D_HID), jnp.float32),
        mesh=mesh,
        compiler_params=pltpu.CompilerParams(use_tc_tiling_on_sc=False),
        scratch_types=[
            pltpu.VMEM((T_G, G_CHUNK), jnp.int32),
            pltpu.VMEM((SUBT, B_IDX), jnp.int32),
            pltpu.VMEM((2, G_CHUNK, D_HID), jnp.float32),
            pltpu.VMEM_SHARED((N_TAB, D_HID), jnp.float32),
            pltpu.SemaphoreType.DMA,
            pltpu.SemaphoreType.DMA,
            pltpu.SemaphoreType.DMA,
            pltpu.SemaphoreType.DMA,
        ],
    )
    return deg, msg


def _mm_body(x_ref, w1_ref, c0_ref, c1_ref, z_ref, dinv_ref):
    deg = 1.0 + c0_ref[...] + c1_ref[...]
    dinv = lax.rsqrt(deg)
    xw = jnp.dot(x_ref[...], w1_ref[...], preferred_element_type=jnp.float32)
    z_ref[...] = xw * dinv
    dinv_ref[...] = dinv


def _mm_call(x, w1, c0, c1):
    grid = (N // R_TC,)
    return pl.pallas_call(
        _mm_body,
        grid=grid,
        in_specs=[
            pl.BlockSpec((R_TC, D_IN), lambda i: (i, 0)),
            pl.BlockSpec((D_IN, D_HID), lambda i: (0, 0)),
            pl.BlockSpec((R_TC, 1), lambda i: (i, 0)),
            pl.BlockSpec((R_TC, 1), lambda i: (i, 0)),
        ],
        out_specs=[
            pl.BlockSpec((R_TC, D_HID), lambda i: (i, 0)),
            pl.BlockSpec((R_TC, 1), lambda i: (i, 0)),
        ],
        out_shape=[
            jax.ShapeDtypeStruct((N, D_HID), jnp.float32),
            jax.ShapeDtypeStruct((N, 1), jnp.float32),
        ],
    )(x, w1, c0, c1)


def _ep_body(a0_ref, a1_ref, z_ref, dinv_ref, b1_ref, w2_ref, b2_ref,
             out_ref):
    t = a0_ref[...] + a1_ref[...] + z_ref[...]
    h = dinv_ref[...] * t + b1_ref[...]
    h = jnp.maximum(h, 0.0)
    y = jnp.dot(h, w2_ref[...], preferred_element_type=jnp.float32)
    y = y + b2_ref[...]
    m = jnp.max(y, axis=-1, keepdims=True)
    e = jnp.exp(y - m)
    lse = jnp.log(jnp.sum(e, axis=-1, keepdims=True)) + m
    out_ref[...] = y - lse


def _ep_call(a0, a1, z, dinv, b1, w2, b2):
    grid = (N // R_TC,)
    rspec = pl.BlockSpec((R_TC, D_HID), lambda i: (i, 0))
    return pl.pallas_call(
        _ep_body,
        grid=grid,
        in_specs=[
            rspec, rspec, rspec,
            pl.BlockSpec((R_TC, 1), lambda i: (i, 0)),
            pl.BlockSpec((1, D_HID), lambda i: (0, 0)),
            pl.BlockSpec((D_HID, D_OUT), lambda i: (0, 0)),
            pl.BlockSpec((1, D_OUT), lambda i: (0, 0)),
        ],
        out_specs=pl.BlockSpec((R_TC, D_OUT), lambda i: (i, 0)),
        out_shape=jax.ShapeDtypeStruct((N, D_OUT), jnp.float32),
    )(a0, a1, z, dinv, b1, w2, b2)


def kernel(x, edge_index, W1, b1, W2, b2):
    src = edge_index[0]
    dst = edge_index[1]
    pad_t = E_TILE - E // NW
    pad_src = jnp.zeros((NW, pad_t), jnp.int32)
    pad_dst = N + (jnp.arange(pad_t, dtype=jnp.int32)[None, :]
                   + (jnp.arange(NW, dtype=jnp.int32)[:, None] % NS) * 15
                   ) % (N_TAB - N)
    src_p = jnp.concatenate([src.reshape(NW, E // NW), pad_src], axis=1)
    dst_p = jnp.concatenate([dst.reshape(NW, E // NW), pad_dst], axis=1)
    src_r = src_p.reshape(NC, NS, T_G, G_CHUNK)
    dst_r = dst_p.reshape(NC, NS, SUBT, B_IDX)

    zeros_n = jnp.zeros((N_TAB,), jnp.float32)
    zeros_nh = jnp.zeros((N_TAB, D_HID), jnp.float32)

    deg_kernel, msg_kernel = _sc_kernels()
    cnt = deg_kernel(dst_r, zeros_n)
    c0 = cnt[0, :N, None]
    c1 = cnt[1, :N, None]
    z, dinv = _mm_call(x, W1, c0, c1)

    acc = msg_kernel(src_r, dst_r, z, zeros_nh)
    a0 = acc[0, :N, :]
    a1 = acc[1, :N, :]
    return _ep_call(a0, a1, z, dinv, b1.reshape(1, D_HID), W2,
                    b2.reshape(1, D_OUT))


# speedup vs baseline: 63.2683x; 1.3274x over previous
import functools

import jax
import jax.numpy as jnp
from jax import lax
from jax.experimental import pallas as pl
from jax.experimental.pallas import tpu as pltpu
from jax.experimental.pallas import tpu_sc as plsc

N = 10000
E = 320000
D_IN = 128
D_HID = 16
D_OUT = 16

NC = 2
NS = 16
NW = NC * NS

B_IDX = 128
G_CHUNK = 1024
T_G = 10
E_TILE = T_G * G_CHUNK
SUBT = E_TILE // B_IDX
SUB = G_CHUNK // B_IDX
T = SUBT
E_PAD = E_TILE * NW
N_TAB = 10240

R_TC = 1000

def _deg_body(dst_hbm, zeros_hbm, cnt_out, idx_v, ones_v, cnt_sh, sem):
    c = lax.axis_index("c")
    s = lax.axis_index("s")

    @pl.when(s == 0)
    def _():
        pltpu.sync_copy(zeros_hbm, cnt_sh)

    pltpu.sync_copy(dst_hbm.at[c, s], idx_v)
    for k in range(B_IDX // 16):
        ones_v[pl.ds(k * 16, 16)] = jnp.full((16,), 1.0, jnp.float32)
    plsc.subcore_barrier()

    def chunk(j, carry):
        pltpu.async_copy(ones_v, cnt_sh.at[idx_v.at[j]], sem, add=True)
        return carry

    lax.fori_loop(0, T, chunk, 0)

    def drain(j, carry):
        pltpu.make_async_copy(ones_v, cnt_sh.at[idx_v.at[j]], sem).wait()
        return carry

    lax.fori_loop(0, T, drain, 0)
    plsc.subcore_barrier()

    @pl.when(s == 0)
    def _():
        pltpu.sync_copy(cnt_sh, cnt_out.at[c])


def _msg_body(src_hbm, dst_hbm, z_hbm, zeros_hbm, acc_out,
              sidx_v, didx_v, rows_v, acc_sh, z_sh, gsem0, gsem1,
              ssem0, ssem1):
    c = lax.axis_index("c")
    s = lax.axis_index("s")

    @pl.when(s == 0)
    def _():
        pltpu.sync_copy(zeros_hbm, acc_sh)

    @pl.when(s == 1)
    def _():
        pltpu.sync_copy(z_hbm, z_sh)

    pltpu.sync_copy(src_hbm.at[c, s], sidx_v)
    pltpu.sync_copy(dst_hbm.at[c, s], didx_v)
    plsc.subcore_barrier()

    gsems = (gsem0, gsem1)
    ssems = (ssem0, ssem1)
    pltpu.async_copy(z_sh.at[sidx_v.at[0]], rows_v.at[0], gsems[0])

    def chunk(g, carry):
        for b in range(2):
            @pl.when(lax.rem(g, 2) == b)
            def _():
                pltpu.make_async_copy(
                    z_sh.at[sidx_v.at[g]], rows_v.at[b], gsems[b]).wait()

                @pl.when(g >= 1)
                def _():
                    pltpu.make_async_copy(
                        z_hbm.at[pl.ds(0, G_CHUNK)], rows_v.at[1 - b],
                        ssems[1 - b]).wait()

                @pl.when(g + 1 < T_G)
                def _():
                    pltpu.async_copy(
                        z_sh.at[sidx_v.at[g + 1]], rows_v.at[1 - b],
                        gsems[1 - b])

                for k in range(SUB):
                    pltpu.async_copy(
                        rows_v.at[b].at[pl.ds(k * B_IDX, B_IDX)],
                        acc_sh.at[didx_v.at[g * SUB + k]], ssems[b],
                        add=True)
        return carry

    lax.fori_loop(0, T_G, chunk, 0)
    last = (T_G - 1) % 2
    pltpu.make_async_copy(z_hbm.at[pl.ds(0, G_CHUNK)], rows_v.at[last],
                          ssems[last]).wait()
    plsc.subcore_barrier()

    @pl.when(s == 0)
    def _():
        pltpu.sync_copy(acc_sh, acc_out.at[c])


@functools.lru_cache(maxsize=None)
def _sc_kernels():
    mesh = plsc.VectorSubcoreMesh(core_axis_name="c", subcore_axis_name="s",
                                  num_cores=NC, num_subcores=NS)
    deg = pl.kernel(
        _deg_body,
        out_type=jax.ShapeDtypeStruct((NC, N_TAB), jnp.float32),
        mesh=mesh,
        scratch_types=[
            pltpu.VMEM((T, B_IDX), jnp.int32),
            pltpu.VMEM((B_IDX,), jnp.float32),
            pltpu.VMEM_SHARED((N_TAB,), jnp.float32),
            pltpu.SemaphoreType.DMA,
        ],
    )
    msg = pl.kernel(
        _msg_body,
        out_type=jax.ShapeDtypeStruct((NC, N_TAB, D_HID), jnp.float32),
        mesh=mesh,
        compiler_params=pltpu.CompilerParams(use_tc_tiling_on_sc=False),
        scratch_types=[
            pltpu.VMEM((T_G, G_CHUNK), jnp.int32),
            pltpu.VMEM((SUBT, B_IDX), jnp.int32),
            pltpu.VMEM((2, G_CHUNK, D_HID), jnp.float32),
            pltpu.VMEM_SHARED((N_TAB, D_HID), jnp.float32),
            pltpu.VMEM_SHARED((N, D_HID), jnp.float32),
            pltpu.SemaphoreType.DMA,
            pltpu.SemaphoreType.DMA,
            pltpu.SemaphoreType.DMA,
            pltpu.SemaphoreType.DMA,
        ],
    )
    return deg, msg


def _mm_body(x_ref, w1_ref, c0_ref, c1_ref, z_ref, dinv_ref):
    deg = 1.0 + c0_ref[...] + c1_ref[...]
    dinv = lax.rsqrt(deg)
    xw = jnp.dot(x_ref[...], w1_ref[...], preferred_element_type=jnp.float32)
    z_ref[...] = xw * dinv
    dinv_ref[...] = dinv


def _mm_call(x, w1, c0, c1):
    grid = (N // R_TC,)
    return pl.pallas_call(
        _mm_body,
        grid=grid,
        in_specs=[
            pl.BlockSpec((R_TC, D_IN), lambda i: (i, 0)),
            pl.BlockSpec((D_IN, D_HID), lambda i: (0, 0)),
            pl.BlockSpec((R_TC, 1), lambda i: (i, 0)),
            pl.BlockSpec((R_TC, 1), lambda i: (i, 0)),
        ],
        out_specs=[
            pl.BlockSpec((R_TC, D_HID), lambda i: (i, 0)),
            pl.BlockSpec((R_TC, 1), lambda i: (i, 0)),
        ],
        out_shape=[
            jax.ShapeDtypeStruct((N, D_HID), jnp.float32),
            jax.ShapeDtypeStruct((N, 1), jnp.float32),
        ],
    )(x, w1, c0, c1)


def _ep_body(a0_ref, a1_ref, z_ref, dinv_ref, b1_ref, w2_ref, b2_ref,
             out_ref):
    t = a0_ref[...] + a1_ref[...] + z_ref[...]
    h = dinv_ref[...] * t + b1_ref[...]
    h = jnp.maximum(h, 0.0)
    y = jnp.dot(h, w2_ref[...], preferred_element_type=jnp.float32)
    y = y + b2_ref[...]
    m = jnp.max(y, axis=-1, keepdims=True)
    e = jnp.exp(y - m)
    lse = jnp.log(jnp.sum(e, axis=-1, keepdims=True)) + m
    out_ref[...] = y - lse


def _ep_call(a0, a1, z, dinv, b1, w2, b2):
    grid = (N // R_TC,)
    rspec = pl.BlockSpec((R_TC, D_HID), lambda i: (i, 0))
    return pl.pallas_call(
        _ep_body,
        grid=grid,
        in_specs=[
            rspec, rspec, rspec,
            pl.BlockSpec((R_TC, 1), lambda i: (i, 0)),
            pl.BlockSpec((1, D_HID), lambda i: (0, 0)),
            pl.BlockSpec((D_HID, D_OUT), lambda i: (0, 0)),
            pl.BlockSpec((1, D_OUT), lambda i: (0, 0)),
        ],
        out_specs=pl.BlockSpec((R_TC, D_OUT), lambda i: (i, 0)),
        out_shape=jax.ShapeDtypeStruct((N, D_OUT), jnp.float32),
    )(a0, a1, z, dinv, b1, w2, b2)


def kernel(x, edge_index, W1, b1, W2, b2):
    src = edge_index[0]
    dst = edge_index[1]
    pad_t = E_TILE - E // NW
    pad_src = jnp.zeros((NW, pad_t), jnp.int32)
    pad_dst = N + (jnp.arange(pad_t, dtype=jnp.int32)[None, :]
                   + (jnp.arange(NW, dtype=jnp.int32)[:, None] % NS) * 15
                   ) % (N_TAB - N)
    src_p = jnp.concatenate([src.reshape(NW, E // NW), pad_src], axis=1)
    dst_p = jnp.concatenate([dst.reshape(NW, E // NW), pad_dst], axis=1)
    src_r = src_p.reshape(NC, NS, T_G, G_CHUNK)
    dst_r = dst_p.reshape(NC, NS, SUBT, B_IDX)

    zeros_n = jnp.zeros((N_TAB,), jnp.float32)
    zeros_nh = jnp.zeros((N_TAB, D_HID), jnp.float32)

    deg_kernel, msg_kernel = _sc_kernels()
    cnt = deg_kernel(dst_r, zeros_n)
    c0 = cnt[0, :N, None]
    c1 = cnt[1, :N, None]
    z, dinv = _mm_call(x, W1, c0, c1)

    acc = msg_kernel(src_r, dst_r, z, zeros_nh)
    a0 = acc[0, :N, :]
    a1 = acc[1, :N, :]
    return _ep_call(a0, a1, z, dinv, b1.reshape(1, D_HID), W2,
                    b2.reshape(1, D_OUT))


# speedup vs baseline: 106.3945x; 1.6816x over previous
import functools

import jax
import jax.numpy as jnp
from jax import lax
from jax.experimental import pallas as pl
from jax.experimental.pallas import tpu as pltpu
from jax.experimental.pallas import tpu_sc as plsc

N = 10000
E = 320000
D_IN = 128
D_HID = 16
D_OUT = 16

NC = 2
NS = 16
NW = NC * NS
E_TILE = E // NW

G_CHUNK = 1000
T_G = E_TILE // G_CHUNK
B_IDX = 128
SUB_F = G_CHUNK // B_IDX
SUB_T = G_CHUNK - SUB_F * B_IDX

DEG_F = E_TILE // B_IDX
DEG_T = E_TILE - DEG_F * B_IDX

N_TAB = 10240
NP = N_TAB // 8
NPR = 1250


def _deg_body(ei_hbm, zeros_hbm, cnt_out, idx_v, ones_v, cnt_sh, sem):
    c = lax.axis_index("c")
    s = lax.axis_index("s")
    w = c * NS + s

    @pl.when(s == 0)
    def _():
        pltpu.sync_copy(zeros_hbm, cnt_sh)

    pltpu.sync_copy(ei_hbm.at[1].at[w], idx_v)
    for k in range(B_IDX // 16):
        ones_v[pl.ds(k * 16, 16)] = jnp.full((16,), 1.0, jnp.float32)
    plsc.subcore_barrier()

    def chunk(j, carry):
        off = pl.multiple_of(j * B_IDX, B_IDX)
        pltpu.async_copy(ones_v, cnt_sh.at[idx_v.at[pl.ds(off, B_IDX)]],
                         sem, add=True)
        return carry

    lax.fori_loop(0, DEG_F, chunk, 0)
    pltpu.async_copy(ones_v.at[pl.ds(0, DEG_T)],
                     cnt_sh.at[idx_v.at[pl.ds(DEG_F * B_IDX, DEG_T)]],
                     sem, add=True)

    def drain(j, carry):
        off = pl.multiple_of(j * B_IDX, B_IDX)
        pltpu.make_async_copy(ones_v,
                              cnt_sh.at[idx_v.at[pl.ds(off, B_IDX)]],
                              sem).wait()
        return carry

    lax.fori_loop(0, DEG_F, drain, 0)
    pltpu.make_async_copy(ones_v.at[pl.ds(0, DEG_T)],
                          cnt_sh.at[idx_v.at[pl.ds(DEG_F * B_IDX, DEG_T)]],
                          sem).wait()
    plsc.subcore_barrier()

    @pl.when(s == 0)
    def _():
        pltpu.sync_copy(cnt_sh, cnt_out.at[c])


def _msg_body(ei_hbm, z_hbm, zeros_hbm, acc_out,
              sidx_v, didx_v, rows_v, acc_sh, z_sh, gsem0, gsem1,
              ssem0, ssem1):
    c = lax.axis_index("c")
    s = lax.axis_index("s")
    w = c * NS + s

    @pl.when(s == 0)
    def _():
        pltpu.sync_copy(zeros_hbm, acc_sh)

    @pl.when(s == 1)
    def _():
        pltpu.sync_copy(z_hbm, z_sh)

    pltpu.sync_copy(ei_hbm.at[0].at[w], sidx_v)
    pltpu.sync_copy(ei_hbm.at[1].at[w], didx_v)
    plsc.subcore_barrier()

    gsems = (gsem0, gsem1)
    ssems = (ssem0, ssem1)
    pltpu.async_copy(z_sh.at[sidx_v.at[pl.ds(0, G_CHUNK)]], rows_v.at[0],
                     gsems[0])

    def chunk(g, carry):
        goff = pl.multiple_of(g * G_CHUNK, G_CHUNK)
        for b in range(2):
            @pl.when(lax.rem(g, 2) == b)
            def _():
                pltpu.make_async_copy(
                    z_sh.at[sidx_v.at[pl.ds(goff, G_CHUNK)]],
                    rows_v.at[b], gsems[b]).wait()

                @pl.when(g >= 1)
                def _():
                    pltpu.make_async_copy(
                        z_hbm.at[pl.ds(0, G_CHUNK)], rows_v.at[1 - b],
                        ssems[1 - b]).wait()

                @pl.when(g + 1 < T_G)
                def _():
                    pltpu.async_copy(
                        z_sh.at[sidx_v.at[pl.ds(goff + G_CHUNK, G_CHUNK)]],
                        rows_v.at[1 - b], gsems[1 - b])

                for k in range(SUB_F):
                    pltpu.async_copy(
                        rows_v.at[b].at[pl.ds(k * B_IDX, B_IDX)],
                        acc_sh.at[didx_v.at[pl.ds(goff + k * B_IDX, B_IDX)]],
                        ssems[b], add=True)
                pltpu.async_copy(
                    rows_v.at[b].at[pl.ds(SUB_F * B_IDX, SUB_T)],
                    acc_sh.at[didx_v.at[pl.ds(goff + SUB_F * B_IDX, SUB_T)]],
                    ssems[b], add=True)
        return carry

    lax.fori_loop(0, T_G, chunk, 0)
    last = (T_G - 1) % 2
    pltpu.make_async_copy(z_hbm.at[pl.ds(0, G_CHUNK)], rows_v.at[last],
                          ssems[last]).wait()
    plsc.subcore_barrier()

    @pl.when(s == 0)
    def _():
        pltpu.sync_copy(acc_sh, acc_out.at[c])


@functools.lru_cache(maxsize=None)
def _sc_kernels():
    mesh = plsc.VectorSubcoreMesh(core_axis_name="c", subcore_axis_name="s",
                                  num_cores=NC, num_subcores=NS)
    deg = pl.kernel(
        _deg_body,
        out_type=jax.ShapeDtypeStruct((NC, N_TAB), jnp.float32),
        mesh=mesh,
        scratch_types=[
            pltpu.VMEM((E_TILE,), jnp.int32),
            pltpu.VMEM((B_IDX,), jnp.float32),
            pltpu.VMEM_SHARED((N_TAB,), jnp.float32),
            pltpu.SemaphoreType.DMA,
        ],
    )
    msg = pl.kernel(
        _msg_body,
        out_type=jax.ShapeDtypeStruct((NC, N_TAB, D_HID), jnp.float32),
        mesh=mesh,
        compiler_params=pltpu.CompilerParams(use_tc_tiling_on_sc=False),
        scratch_types=[
            pltpu.VMEM((E_TILE,), jnp.int32),
            pltpu.VMEM((E_TILE,), jnp.int32),
            pltpu.VMEM((2, G_CHUNK, D_HID), jnp.float32),
            pltpu.VMEM_SHARED((N_TAB, D_HID), jnp.float32),
            pltpu.VMEM_SHARED((N_TAB, D_HID), jnp.float32),
            pltpu.SemaphoreType.DMA,
            pltpu.SemaphoreType.DMA,
            pltpu.SemaphoreType.DMA,
            pltpu.SemaphoreType.DMA,
        ],
    )
    return deg, msg


def _mm_body(x3_ref, w1_ref, cnt_ref, p_ref, z_ref, dinv_ref, d3_ref):
    cn = cnt_ref[0] + cnt_ref[1]
    for u in range(16):
        d3_ref[:, u, :] = jnp.dot(cn, p_ref[u],
                                  preferred_element_type=jnp.float32)
    dinv3 = lax.rsqrt(1.0 + d3_ref[...])
    dinv_ref[...] = dinv3
    dinv128 = dinv3.reshape(NP, 128)
    z_ref[pl.ds(NPR, NP - NPR), :] = jnp.zeros((NP - NPR, 128), jnp.float32)
    for g in range(8):
        y = jnp.dot(x3_ref[:, g, :], w1_ref[...],
                    preferred_element_type=jnp.float32)
        z_ref[pl.ds(0, NPR), pl.ds(g * 16, 16)] = (
            y * dinv128[0:NPR, g * 16:(g + 1) * 16])


def _mm_call(x3, w1, cnt128, p):
    return pl.pallas_call(
        _mm_body,
        grid=(1,),
        in_specs=[
            pl.BlockSpec((NPR, 8, D_IN), lambda i: (0, 0, 0)),
            pl.BlockSpec((D_IN, D_HID), lambda i: (0, 0)),
            pl.BlockSpec((NC, 80, 128), lambda i: (0, 0, 0)),
            pl.BlockSpec((16, 128, 128), lambda i: (0, 0, 0)),
        ],
        out_specs=[
            pl.BlockSpec((NP, 128), lambda i: (0, 0)),
            pl.BlockSpec((80, 16, 128), lambda i: (0, 0, 0)),
        ],
        out_shape=[
            jax.ShapeDtypeStruct((NP, 128), jnp.float32),
            jax.ShapeDtypeStruct((80, 16, 128), jnp.float32),
        ],
        scratch_shapes=[pltpu.VMEM((80, 16, 128), jnp.float32)],
    )(x3, w1, cnt128, p)


def _ep_body(acc_ref, z_ref, dinv_ref, b1_ref, w2bd_ref, b2_ref,
             onesbd_ref, out_ref):
    t = acc_ref[0] + acc_ref[1] + z_ref[...]
    h = dinv_ref[...] * t + b1_ref[...]
    h = jnp.maximum(h, 0.0)
    y = jnp.dot(h, w2bd_ref[...], preferred_element_type=jnp.float32)
    y = y + b2_ref[...]
    e = jnp.exp(y)
    ssum = jnp.dot(e, onesbd_ref[...], preferred_element_type=jnp.float32)
    out_ref[...] = y - jnp.log(ssum)


def _ep_call(acc128, z128, dinv128, b1t, w2bd, b2t, onesbd):
    return pl.pallas_call(
        _ep_body,
        grid=(1,),
        in_specs=[
            pl.BlockSpec((NC, NP, 128), lambda i: (0, 0, 0)),
            pl.BlockSpec((NP, 128), lambda i: (0, 0)),
            pl.BlockSpec((NP, 128), lambda i: (0, 0)),
            pl.BlockSpec((1, 128), lambda i: (0, 0)),
            pl.BlockSpec((128, 128), lambda i: (0, 0)),
            pl.BlockSpec((1, 128), lambda i: (0, 0)),
            pl.BlockSpec((128, 128), lambda i: (0, 0)),
        ],
        out_specs=pl.BlockSpec((NP, 128), lambda i: (0, 0)),
        out_shape=jax.ShapeDtypeStruct((NP, 128), jnp.float32),
    )(acc128, z128, dinv128, b1t, w2bd, b2t, onesbd)


def kernel(x, edge_index, W1, b1, W2, b2):
    zeros_n = jnp.zeros((N_TAB,), jnp.float32)
    zeros_nh = jnp.zeros((N_TAB, D_HID), jnp.float32)

    uu = jnp.arange(16, dtype=jnp.int32)[:, None, None]
    mm = jnp.arange(128, dtype=jnp.int32)[None, :, None]
    ll = jnp.arange(128, dtype=jnp.int32)[None, None, :]
    p = (mm == 8 * uu + ll // 16).astype(jnp.float32)

    eye8 = jnp.eye(8, dtype=jnp.float32)
    w2bd = jnp.kron(eye8, W2)
    onesbd = jnp.kron(eye8, jnp.ones((D_HID, D_OUT), jnp.float32))
    b1t = jnp.tile(b1, 8).reshape(1, 128)
    b2t = jnp.tile(b2, 8).reshape(1, 128)

    deg_kernel, msg_kernel = _sc_kernels()
    ei3 = edge_index.reshape(2, NW, E_TILE)
    cnt = deg_kernel(ei3, zeros_n)
    cnt128 = cnt.reshape(NC, 80, 128)

    x3 = x.reshape(NPR, 8, D_IN)
    z128, dinv3 = _mm_call(x3, W1, cnt128, p)

    z16 = z128.reshape(N_TAB, D_HID)
    acc = msg_kernel(ei3, z16, zeros_nh)

    acc128 = acc.reshape(NC, NP, 128)
    dinv128 = dinv3.reshape(NP, 128)
    out128 = _ep_call(acc128, z128, dinv128, b1t, w2bd, b2t, onesbd)
    return out128.reshape(N_TAB, D_HID)[:N]
